# baseline (speedup 1.0000x reference)
import functools

import jax
import jax.numpy as jnp
from jax import lax
from jax.experimental import pallas as pl
from jax.experimental.pallas import tpu as pltpu
from jax.experimental.pallas import tpu_sc as plsc

_V = 10000
_E = 160000
_K = 4
_H = 64
_OUT = 128
_L = 3
_F = 2 * _H

_NSUB = 16
_CHUNK = 128
_CPT = 79
_GROUP = 56
_EPAD = _NSUB * _CPT * _CHUNK
_VPAD = 10112
_RPT = _VPAD // _NSUB


def _sc_agg_body(x0, x1, x2, x3, srcp, dstp, o0, o1, o2, o3,
                 acc, src_v, dst_v, buf0, buf1, sg0, sg1, ss0, ss1):
    c = lax.axis_index("c")
    s = lax.axis_index("s")
    xs = (x0, x1, x2, x3)
    outs = (o0, o1, o2, o3)
    zero = jnp.zeros((16,), jnp.float32)

    def zrow(i, carry):
        for j in range(_F // 16):
            buf0[i, pl.ds(j * 16, 16)] = zero
        return carry

    row0 = s * _RPT
    nz_full = _RPT // _CHUNK
    nz_tail = _RPT - nz_full * _CHUNK
    groups = []
    g0 = 0
    while g0 < _CPT:
        groups.append((g0, min(_GROUP, _CPT - g0)))
        g0 += _GROUP
    for b in range(4):
        @pl.when(c == b // 2)
        def _(b=b):
            lax.fori_loop(0, _CHUNK, zrow, 0)
            for q in range(nz_full):
                pltpu.sync_copy(buf0, acc.at[pl.ds(row0 + q * _CHUNK, _CHUNK)])
            if nz_tail:
                pltpu.sync_copy(buf0.at[pl.ds(0, nz_tail)],
                                acc.at[pl.ds(row0 + nz_full * _CHUNK, nz_tail)])
            plsc.subcore_barrier()

            def gstart(j, buf, sem):
                pltpu.async_copy(xs[b].at[src_v.at[j]], buf, sem)

            def gwait(j, buf, sem):
                pltpu.make_async_copy(xs[b].at[src_v.at[j]], buf, sem).wait()

            for gbase, glen in groups:
                pltpu.sync_copy(srcp.at[s].at[pl.ds(gbase, glen)],
                                src_v.at[pl.ds(0, glen)])
                pltpu.sync_copy(dstp.at[s].at[pl.ds(gbase, glen)],
                                dst_v.at[pl.ds(0, glen)])
                npair = glen // 2
                gstart(0, buf0, sg0)

                def swait(buf, j, sem):
                    pltpu.make_async_copy(buf, acc.at[dst_v.at[j]],
                                          sem).wait()

                def pair(g, carry):
                    j0 = 2 * g
                    j1 = j0 + 1

                    @pl.when(g > 0)
                    def _():
                        swait(buf1, j1, ss1)

                    gstart(j1, buf1, sg1)
                    gwait(j0, buf0, sg0)
                    pltpu.async_copy(buf0, acc.at[dst_v.at[j0]], ss0,
                                     add=True)
                    gwait(j1, buf1, sg1)
                    swait(buf0, j0, ss0)

                    @pl.when(g < npair - 1)
                    def _():
                        gstart(j0 + 2, buf0, sg0)

                    pltpu.async_copy(buf1, acc.at[dst_v.at[j1]], ss1,
                                     add=True)
                    return carry

                lax.fori_loop(0, npair, pair, 0)
                swait(buf1, 2 * npair - 1, ss1)
                if glen % 2:
                    jt = glen - 1
                    pltpu.async_copy(xs[b].at[src_v.at[jt]], buf0, sg0).wait()
                    pltpu.sync_copy(buf0, acc.at[dst_v.at[jt]], add=True)
            plsc.subcore_barrier()
            pltpu.sync_copy(acc.at[pl.ds(row0, _RPT)],
                            outs[b].at[pl.ds(row0, _RPT)])


@functools.lru_cache(maxsize=None)
def _get_sc_agg():
    return pl.kernel(
        _sc_agg_body,
        out_type=tuple(jax.ShapeDtypeStruct((_VPAD, _F), jnp.float32)
                       for _ in range(4)),
        mesh=plsc.VectorSubcoreMesh(core_axis_name="c", subcore_axis_name="s",
                                    num_cores=2, num_subcores=_NSUB),
        scratch_types=[
            pltpu.VMEM_SHARED((_VPAD, _F), jnp.float32),
            pltpu.VMEM((_GROUP, _CHUNK), jnp.int32),
            pltpu.VMEM((_GROUP, _CHUNK), jnp.int32),
            pltpu.VMEM((_CHUNK, _F), jnp.float32),
            pltpu.VMEM((_CHUNK, _F), jnp.float32),
            pltpu.SemaphoreType.DMA,
            pltpu.SemaphoreType.DMA,
            pltpu.SemaphoreType.DMA,
            pltpu.SemaphoreType.DMA,
        ],
    )


_R0 = 1000
_R1 = 2000


def _x0_body(spec, w0, w1, bias, o0, o1, o2, o3):
    outs = (o0, o1, o2, o3)
    for k in range(_K):
        e = spec[:, k:k + 1]
        f = spec[:, _K + k:_K + k + 1]
        a = e * w0[...] + bias[...]
        d = f * w1[...]
        outs[k][...] = jnp.concatenate([a + d, a - d], axis=1)


_x0_call = pl.pallas_call(
    _x0_body,
    grid=(_V // _R0,),
    in_specs=[
        pl.BlockSpec((_R0, 2 * _K), lambda i: (i, 0)),
        pl.BlockSpec((1, _H), lambda i: (0, 0)),
        pl.BlockSpec((1, _H), lambda i: (0, 0)),
        pl.BlockSpec((1, _H), lambda i: (0, 0)),
    ],
    out_specs=[pl.BlockSpec((_R0, _F), lambda i: (i, 0)) for _ in range(4)],
    out_shape=[jax.ShapeDtypeStruct((_V, _F), jnp.float32) for _ in range(4)],
)


def _gin_body(scale, x0, x1, x2, x3, a0, a1, a2, a3, w1, b1, w2, b2,
              o0, o1, o2, o3):
    xs = (x0, x1, x2, x3)
    ags = (a0, a1, a2, a3)
    outs = (o0, o1, o2, o3)
    sc = scale[0]
    for k in range(_K):
        t = sc * xs[k][...] + ags[k][...]
        tl = t[:, :_H]
        tr = t[:, _H:]
        hl = jnp.maximum(
            jnp.dot(tl, w1[...], preferred_element_type=jnp.float32)
            + b1[...], 0.0)
        hr = jnp.maximum(
            jnp.dot(tr, w1[...], preferred_element_type=jnp.float32)
            + b1[...], 0.0)
        ol = jnp.dot(hl, w2[...], preferred_element_type=jnp.float32) + b2[...]
        orr = jnp.dot(hr, w2[...], preferred_element_type=jnp.float32) + b2[...]
        outs[k][...] = jnp.concatenate([ol, orr], axis=1)


_gin_call = pl.pallas_call(
    _gin_body,
    grid=(_V // _R1,),
    in_specs=[
        pl.BlockSpec(memory_space=pltpu.SMEM),
    ] + [pl.BlockSpec((_R1, _F), lambda i: (i, 0)) for _ in range(8)] + [
        pl.BlockSpec((_H, _H), lambda i: (0, 0)),
        pl.BlockSpec((1, _H), lambda i: (0, 0)),
        pl.BlockSpec((_H, _H), lambda i: (0, 0)),
        pl.BlockSpec((1, _H), lambda i: (0, 0)),
    ],
    out_specs=[pl.BlockSpec((_R1, _F), lambda i: (i, 0)) for _ in range(4)],
    out_shape=[jax.ShapeDtypeStruct((_V, _F), jnp.float32) for _ in range(4)],
)


def _merge_body(*refs):
    xs = refs[:12]
    (scale, a0, a1, a2, a3, w1, b1, w2, b2,
     skw, skb, mw1, mb1, mw2, mb2, mw3, mb3, out) = refs[12:]
    ags = (a0, a1, a2, a3)
    sc = scale[0]
    zs = []
    for k in range(_K):
        x2 = xs[2 * 4 + k]
        t = sc * x2[...] + ags[k][...]
        x3_halves = []
        for h0 in (0, _H):
            hh = jnp.maximum(
                jnp.dot(t[:, h0:h0 + _H], w1[...],
                        preferred_element_type=jnp.float32) + b1[...], 0.0)
            x3_halves.append(
                jnp.dot(hh, w2[...], preferred_element_type=jnp.float32)
                + b2[...])
        accum = None
        for l in range(_L + 1):
            if l < _L:
                xr = xs[l * 4 + k]
                srow = xr[:, :_H] + xr[:, _H:]
            else:
                srow = x3_halves[0] + x3_halves[1]
            t = jnp.dot(srow, skw[l * _H:(l + 1) * _H, :],
                        preferred_element_type=jnp.float32)
            accum = t if accum is None else accum + t
        zs.append(accum + 2.0 * skb[...])
    z = jnp.concatenate(zs, axis=1)
    y = jnp.maximum(
        jnp.dot(z, mw1[...], preferred_element_type=jnp.float32) + mb1[...],
        0.0)
    y = jnp.maximum(
        jnp.dot(y, mw2[...], preferred_element_type=jnp.float32) + mb2[...],
        0.0)
    out[...] = jnp.dot(y, mw3[...], preferred_element_type=jnp.float32) + mb3[...]


_merge_call = pl.pallas_call(
    _merge_body,
    grid=(_V // _R0,),
    in_specs=[pl.BlockSpec((_R0, _F), lambda i: (i, 0)) for _ in range(12)] + [
        pl.BlockSpec(memory_space=pltpu.SMEM),
    ] + [pl.BlockSpec((_R0, _F), lambda i: (i, 0)) for _ in range(4)] + [
        pl.BlockSpec((_H, _H), lambda i: (0, 0)),
        pl.BlockSpec((1, _H), lambda i: (0, 0)),
        pl.BlockSpec((_H, _H), lambda i: (0, 0)),
        pl.BlockSpec((1, _H), lambda i: (0, 0)),
    ] + [
        pl.BlockSpec(((_L + 1) * _H, _H), lambda i: (0, 0)),
        pl.BlockSpec((1, _H), lambda i: (0, 0)),
        pl.BlockSpec((_K * _H, _H), lambda i: (0, 0)),
        pl.BlockSpec((1, _H), lambda i: (0, 0)),
        pl.BlockSpec((_H, _H), lambda i: (0, 0)),
        pl.BlockSpec((1, _H), lambda i: (0, 0)),
        pl.BlockSpec((_H, _OUT), lambda i: (0, 0)),
        pl.BlockSpec((1, _OUT), lambda i: (0, 0)),
    ],
    out_specs=pl.BlockSpec((_R0, _OUT), lambda i: (i, 0)),
    out_shape=jax.ShapeDtypeStruct((_V, _OUT), jnp.float32),
)


def kernel(spectral_features, edge_index, in_W, in_b, conv_eps, conv_W1,
           conv_b1, conv_W2, conv_b2, skip_W, skip_b, merge_W1, merge_b1,
           merge_W2, merge_b2, merge_W3, merge_b3):
    x = _x0_call(spectral_features, in_W[0:1, :], in_W[1:2, :],
                 in_b.reshape(1, _H))
    pad = _EPAD - _E
    srcp = jnp.concatenate(
        [edge_index[0], jnp.zeros((pad,), jnp.int32)]).reshape(
            _NSUB, _CPT, _CHUNK)
    dstp = jnp.concatenate(
        [edge_index[1], jnp.full((pad,), _V, jnp.int32)]).reshape(
            _NSUB, _CPT, _CHUNK)
    xs_all = [x]
    for l in range(_L - 1):
        aggs = _get_sc_agg()(x[0], x[1], x[2], x[3], srcp, dstp)
        scale = (1.0 + conv_eps[l]).astype(jnp.float32).reshape(1)
        x = _gin_call(scale, *x, *aggs, conv_W1[l],
                      conv_b1[l].reshape(1, _H), conv_W2[l],
                      conv_b2[l].reshape(1, _H))
        xs_all.append(x)
    aggs = _get_sc_agg()(x[0], x[1], x[2], x[3], srcp, dstp)
    scale = (1.0 + conv_eps[_L - 1]).astype(jnp.float32).reshape(1)
    flat = [xs_all[l][k] for l in range(_L) for k in range(_K)]
    return _merge_call(*flat, scale, *aggs,
                       conv_W1[_L - 1], conv_b1[_L - 1].reshape(1, _H),
                       conv_W2[_L - 1], conv_b2[_L - 1].reshape(1, _H),
                       skip_W, skip_b.reshape(1, _H),
                       merge_W1, merge_b1.reshape(1, _H),
                       merge_W2, merge_b2.reshape(1, _H),
                       merge_W3, merge_b3.reshape(1, _OUT))


# speedup vs baseline: 1.0122x; 1.0122x over previous
import functools

import jax
import jax.numpy as jnp
from jax import lax
from jax.experimental import pallas as pl
from jax.experimental.pallas import tpu as pltpu
from jax.experimental.pallas import tpu_sc as plsc

_V = 10000
_E = 160000
_K = 4
_H = 64
_OUT = 128
_L = 3
_F = 2 * _H

_NSUB = 16
_CHUNK = 128
_CPT = 79
_GROUP = 56
_EPAD = _NSUB * _CPT * _CHUNK
_VPAD = 10112
_RPT = _VPAD // _NSUB


def _sc_agg_body(x0, x1, x2, x3, srcp, dstp, o0, o1, o2, o3,
                 acc, src_v, dst_v, buf0, buf1, sg0, sg1, ss0, ss1):
    c = lax.axis_index("c")
    s = lax.axis_index("s")
    xs = (x0, x1, x2, x3)
    outs = (o0, o1, o2, o3)
    zero = jnp.zeros((16,), jnp.float32)

    def zrow(i, carry):
        for j in range(_F // 16):
            buf0[i, pl.ds(j * 16, 16)] = zero
        return carry

    row0 = s * _RPT
    nz_full = _RPT // _CHUNK
    nz_tail = _RPT - nz_full * _CHUNK
    groups = []
    g0 = 0
    while g0 < _CPT:
        groups.append((g0, min(_GROUP, _CPT - g0)))
        g0 += _GROUP
    for b in range(4):
        @pl.when(c == b // 2)
        def _(b=b):
            lax.fori_loop(0, _CHUNK, zrow, 0)
            for q in range(nz_full):
                pltpu.sync_copy(buf0, acc.at[pl.ds(row0 + q * _CHUNK, _CHUNK)])
            if nz_tail:
                pltpu.sync_copy(buf0.at[pl.ds(0, nz_tail)],
                                acc.at[pl.ds(row0 + nz_full * _CHUNK, nz_tail)])
            plsc.subcore_barrier()

            def gstart(j, buf, sem):
                pltpu.async_copy(xs[b].at[src_v.at[j]], buf, sem)

            def gwait(j, buf, sem):
                pltpu.make_async_copy(xs[b].at[src_v.at[j]], buf, sem).wait()

            for gbase, glen in groups:
                pltpu.sync_copy(srcp.at[s].at[pl.ds(gbase, glen)],
                                src_v.at[pl.ds(0, glen)])
                pltpu.sync_copy(dstp.at[s].at[pl.ds(gbase, glen)],
                                dst_v.at[pl.ds(0, glen)])
                npair = glen // 2
                gstart(0, buf0, sg0)

                def swait(buf, j, sem):
                    pltpu.make_async_copy(buf, acc.at[dst_v.at[j]],
                                          sem).wait()

                def pair(g, carry):
                    j0 = 2 * g
                    j1 = j0 + 1

                    @pl.when(g > 0)
                    def _():
                        swait(buf1, j1, ss1)

                    gstart(j1, buf1, sg1)
                    gwait(j0, buf0, sg0)
                    pltpu.async_copy(buf0, acc.at[dst_v.at[j0]], ss0,
                                     add=True)
                    gwait(j1, buf1, sg1)
                    swait(buf0, j0, ss0)

                    @pl.when(g < npair - 1)
                    def _():
                        gstart(j0 + 2, buf0, sg0)

                    pltpu.async_copy(buf1, acc.at[dst_v.at[j1]], ss1,
                                     add=True)
                    return carry

                lax.fori_loop(0, npair, pair, 0)
                swait(buf1, 2 * npair - 1, ss1)
                if glen % 2:
                    jt = glen - 1
                    pltpu.async_copy(xs[b].at[src_v.at[jt]], buf0, sg0).wait()
                    pltpu.sync_copy(buf0, acc.at[dst_v.at[jt]], add=True)
            plsc.subcore_barrier()
            pltpu.sync_copy(acc.at[pl.ds(row0, _RPT)],
                            outs[b].at[pl.ds(row0, _RPT)])


@functools.lru_cache(maxsize=None)
def _get_sc_agg():
    return pl.kernel(
        _sc_agg_body,
        out_type=tuple(jax.ShapeDtypeStruct((_VPAD, _F), jnp.float32)
                       for _ in range(4)),
        mesh=plsc.VectorSubcoreMesh(core_axis_name="c", subcore_axis_name="s",
                                    num_cores=2, num_subcores=_NSUB),
        scratch_types=[
            pltpu.VMEM_SHARED((_VPAD, _F), jnp.float32),
            pltpu.VMEM((_GROUP, _CHUNK), jnp.int32),
            pltpu.VMEM((_GROUP, _CHUNK), jnp.int32),
            pltpu.VMEM((_CHUNK, _F), jnp.float32),
            pltpu.VMEM((_CHUNK, _F), jnp.float32),
            pltpu.SemaphoreType.DMA,
            pltpu.SemaphoreType.DMA,
            pltpu.SemaphoreType.DMA,
            pltpu.SemaphoreType.DMA,
        ],
    )


_R0 = 2000
_R1 = 2000


def _x0_body(spec, w0, w1, bias, o0, o1, o2, o3):
    outs = (o0, o1, o2, o3)
    for k in range(_K):
        e = spec[:, k:k + 1]
        f = spec[:, _K + k:_K + k + 1]
        a = e * w0[...] + bias[...]
        d = f * w1[...]
        outs[k][...] = jnp.concatenate([a + d, a - d], axis=1)


_x0_call = pl.pallas_call(
    _x0_body,
    grid=(_V // _R0,),
    in_specs=[
        pl.BlockSpec((_R0, 2 * _K), lambda i: (i, 0)),
        pl.BlockSpec((1, _H), lambda i: (0, 0)),
        pl.BlockSpec((1, _H), lambda i: (0, 0)),
        pl.BlockSpec((1, _H), lambda i: (0, 0)),
    ],
    out_specs=[pl.BlockSpec((_R0, _F), lambda i: (i, 0)) for _ in range(4)],
    out_shape=[jax.ShapeDtypeStruct((_V, _F), jnp.float32) for _ in range(4)],
)


def _gin_body(scale, x0, x1, x2, x3, a0, a1, a2, a3, w1, b1, w2, b2,
              o0, o1, o2, o3):
    xs = (x0, x1, x2, x3)
    ags = (a0, a1, a2, a3)
    outs = (o0, o1, o2, o3)
    sc = scale[0]
    for k in range(_K):
        t = sc * xs[k][...] + ags[k][...]
        tl = t[:, :_H]
        tr = t[:, _H:]
        hl = jnp.maximum(
            jnp.dot(tl, w1[...], preferred_element_type=jnp.float32)
            + b1[...], 0.0)
        hr = jnp.maximum(
            jnp.dot(tr, w1[...], preferred_element_type=jnp.float32)
            + b1[...], 0.0)
        ol = jnp.dot(hl, w2[...], preferred_element_type=jnp.float32) + b2[...]
        orr = jnp.dot(hr, w2[...], preferred_element_type=jnp.float32) + b2[...]
        outs[k][...] = jnp.concatenate([ol, orr], axis=1)


_gin_call = pl.pallas_call(
    _gin_body,
    grid=(_V // _R1,),
    in_specs=[
        pl.BlockSpec(memory_space=pltpu.SMEM),
    ] + [pl.BlockSpec((_R1, _F), lambda i: (i, 0)) for _ in range(8)] + [
        pl.BlockSpec((_H, _H), lambda i: (0, 0)),
        pl.BlockSpec((1, _H), lambda i: (0, 0)),
        pl.BlockSpec((_H, _H), lambda i: (0, 0)),
        pl.BlockSpec((1, _H), lambda i: (0, 0)),
    ],
    out_specs=[pl.BlockSpec((_R1, _F), lambda i: (i, 0)) for _ in range(4)],
    out_shape=[jax.ShapeDtypeStruct((_V, _F), jnp.float32) for _ in range(4)],
)


def _merge_body(*refs):
    xs = refs[:12]
    (scale, a0, a1, a2, a3, w1, b1, w2, b2,
     skw, skb, mw1, mb1, mw2, mb2, mw3, mb3, out) = refs[12:]
    ags = (a0, a1, a2, a3)
    sc = scale[0]
    zs = []
    for k in range(_K):
        x2 = xs[2 * 4 + k]
        t = sc * x2[...] + ags[k][...]
        x3_halves = []
        for h0 in (0, _H):
            hh = jnp.maximum(
                jnp.dot(t[:, h0:h0 + _H], w1[...],
                        preferred_element_type=jnp.float32) + b1[...], 0.0)
            x3_halves.append(
                jnp.dot(hh, w2[...], preferred_element_type=jnp.float32)
                + b2[...])
        accum = None
        for l in range(_L + 1):
            if l < _L:
                xr = xs[l * 4 + k]
                srow = xr[:, :_H] + xr[:, _H:]
            else:
                srow = x3_halves[0] + x3_halves[1]
            t = jnp.dot(srow, skw[l * _H:(l + 1) * _H, :],
                        preferred_element_type=jnp.float32)
            accum = t if accum is None else accum + t
        zs.append(accum + 2.0 * skb[...])
    z = jnp.concatenate(zs, axis=1)
    y = jnp.maximum(
        jnp.dot(z, mw1[...], preferred_element_type=jnp.float32) + mb1[...],
        0.0)
    y = jnp.maximum(
        jnp.dot(y, mw2[...], preferred_element_type=jnp.float32) + mb2[...],
        0.0)
    out[...] = jnp.dot(y, mw3[...], preferred_element_type=jnp.float32) + mb3[...]


_merge_call = pl.pallas_call(
    _merge_body,
    grid=(_V // _R0,),
    in_specs=[pl.BlockSpec((_R0, _F), lambda i: (i, 0)) for _ in range(12)] + [
        pl.BlockSpec(memory_space=pltpu.SMEM),
    ] + [pl.BlockSpec((_R0, _F), lambda i: (i, 0)) for _ in range(4)] + [
        pl.BlockSpec((_H, _H), lambda i: (0, 0)),
        pl.BlockSpec((1, _H), lambda i: (0, 0)),
        pl.BlockSpec((_H, _H), lambda i: (0, 0)),
        pl.BlockSpec((1, _H), lambda i: (0, 0)),
    ] + [
        pl.BlockSpec(((_L + 1) * _H, _H), lambda i: (0, 0)),
        pl.BlockSpec((1, _H), lambda i: (0, 0)),
        pl.BlockSpec((_K * _H, _H), lambda i: (0, 0)),
        pl.BlockSpec((1, _H), lambda i: (0, 0)),
        pl.BlockSpec((_H, _H), lambda i: (0, 0)),
        pl.BlockSpec((1, _H), lambda i: (0, 0)),
        pl.BlockSpec((_H, _OUT), lambda i: (0, 0)),
        pl.BlockSpec((1, _OUT), lambda i: (0, 0)),
    ],
    out_specs=pl.BlockSpec((_R0, _OUT), lambda i: (i, 0)),
    out_shape=jax.ShapeDtypeStruct((_V, _OUT), jnp.float32),
)


def kernel(spectral_features, edge_index, in_W, in_b, conv_eps, conv_W1,
           conv_b1, conv_W2, conv_b2, skip_W, skip_b, merge_W1, merge_b1,
           merge_W2, merge_b2, merge_W3, merge_b3):
    x = _x0_call(spectral_features, in_W[0:1, :], in_W[1:2, :],
                 in_b.reshape(1, _H))
    pad = _EPAD - _E
    srcp = jnp.concatenate(
        [edge_index[0], jnp.zeros((pad,), jnp.int32)]).reshape(
            _NSUB, _CPT, _CHUNK)
    dstp = jnp.concatenate(
        [edge_index[1], jnp.full((pad,), _V, jnp.int32)]).reshape(
            _NSUB, _CPT, _CHUNK)
    xs_all = [x]
    for l in range(_L - 1):
        aggs = _get_sc_agg()(x[0], x[1], x[2], x[3], srcp, dstp)
        scale = (1.0 + conv_eps[l]).astype(jnp.float32).reshape(1)
        x = _gin_call(scale, *x, *aggs, conv_W1[l],
                      conv_b1[l].reshape(1, _H), conv_W2[l],
                      conv_b2[l].reshape(1, _H))
        xs_all.append(x)
    aggs = _get_sc_agg()(x[0], x[1], x[2], x[3], srcp, dstp)
    scale = (1.0 + conv_eps[_L - 1]).astype(jnp.float32).reshape(1)
    flat = [xs_all[l][k] for l in range(_L) for k in range(_K)]
    return _merge_call(*flat, scale, *aggs,
                       conv_W1[_L - 1], conv_b1[_L - 1].reshape(1, _H),
                       conv_W2[_L - 1], conv_b2[_L - 1].reshape(1, _H),
                       skip_W, skip_b.reshape(1, _H),
                       merge_W1, merge_b1.reshape(1, _H),
                       merge_W2, merge_b2.reshape(1, _H),
                       merge_W3, merge_b3.reshape(1, _OUT))
